# Initial kernel scaffold; baseline (speedup 1.0000x reference)
import jax
import jax.numpy as jnp
from jax.experimental import pallas as pl


def kernel(x, edge_index, edge_weight, W_in, b_in, W_enc, b_enc, W_bot, b_bot, W_dec, b_dec, W_out, b_out):
    raise NotImplementedError("write your pallas kernel here")


# speedup vs baseline: 2.9401x; 2.9401x over previous
import functools

import jax
import jax.numpy as jnp
from jax import lax
from jax.experimental import pallas as pl
from jax.experimental.pallas import tpu as pltpu
from jax.experimental.pallas import tpu_sc as plsc

NC = 2
NS = 16
LANES = 16
NW = NC * NS

CHUNK = 128
SB = 16
NB = 5


def _sc_mesh():
    return plsc.VectorSubcoreMesh(
        core_axis_name="c", subcore_axis_name="s", num_cores=NC, num_subcores=NS
    )


@functools.lru_cache(maxsize=None)
def _make_deg_kernel(n_pad, nb, sb, c):
    rpt = n_pad // NS
    zb = 8

    @functools.partial(
        pl.kernel,
        out_type=jax.ShapeDtypeStruct((NC, n_pad, LANES), jnp.float32),
        mesh=_sc_mesh(),
        scratch_types=[
            pltpu.VMEM((sb, c), jnp.int32),
            pltpu.VMEM((sb, c), jnp.float32),
            pltpu.VMEM((c, LANES), jnp.float32),
            pltpu.VMEM((zb, LANES), jnp.float32),
            pltpu.VMEM_SHARED((n_pad, LANES), jnp.float32),
            pltpu.SemaphoreType.DMA,
        ],
    )
    def deg_kernel(dst_hbm, ew_hbm, out_hbm, dst_v, ew_v, rep_v, zero_v, acc_sh, sem):
        del sem
        cid = lax.axis_index("c")
        sid = lax.axis_index("s")
        wid = sid * NC + cid

        def _zero(i, _):
            zero_v[i] = jnp.zeros((LANES,), jnp.float32)
            return 0

        lax.fori_loop(0, zb, _zero, 0)

        def _zcopy(k, _):
            pltpu.sync_copy(zero_v, acc_sh.at[pl.ds(sid * rpt + k * zb, zb)])
            return 0

        lax.fori_loop(0, rpt // zb, _zcopy, 0)
        plsc.subcore_barrier()

        def block(b, _):
            pltpu.sync_copy(dst_hbm.at[wid, b], dst_v)
            pltpu.sync_copy(ew_hbm.at[wid, b], ew_v)

            def chunk(j, _):
                def group(g, _):
                    ew16 = ew_v[j, pl.ds(g * LANES, LANES)]
                    for l in range(LANES):
                        rep_v[g * LANES + l] = jnp.full((LANES,), ew16[l], jnp.float32)
                    return 0

                lax.fori_loop(0, c // LANES, group, 0)
                pltpu.sync_copy(rep_v, acc_sh.at[dst_v.at[j]], add=True)
                return 0

            lax.fori_loop(0, sb, chunk, 0)
            return 0

        lax.fori_loop(0, nb, block, 0)
        plsc.subcore_barrier()
        pltpu.sync_copy(
            acc_sh.at[pl.ds(sid * rpt, rpt)],
            out_hbm.at[cid].at[pl.ds(sid * rpt, rpt)],
        )

    return deg_kernel


@functools.lru_cache(maxsize=None)
def _make_agg_kernel(n_pad, d, nb, sb, c):
    rpt = n_pad // NS
    zb = 32

    @functools.partial(
        pl.kernel,
        out_type=jax.ShapeDtypeStruct((NC, n_pad, d), jnp.float32),
        mesh=_sc_mesh(),
        scratch_types=[
            pltpu.VMEM((sb, c), jnp.int32),
            pltpu.VMEM((sb, c), jnp.int32),
            pltpu.VMEM((sb, c), jnp.float32),
            pltpu.VMEM((c, d), jnp.float32),
            pltpu.VMEM((zb, d), jnp.float32),
            pltpu.VMEM_SHARED((n_pad, d), jnp.float32),
            pltpu.SemaphoreType.DMA,
        ],
    )
    def agg_kernel(
        h_hbm, src_hbm, dst_hbm, ew_hbm, out_hbm,
        src_v, dst_v, ew_v, rows_v, zero_v, acc_sh, sem,
    ):
        cid = lax.axis_index("c")
        sid = lax.axis_index("s")
        wid = sid * NC + cid
        nvec = d // LANES

        def _zero(i, _):
            r = i // nvec
            col = (i % nvec) * LANES
            zero_v[r, pl.ds(col, LANES)] = jnp.zeros((LANES,), jnp.float32)
            return 0

        lax.fori_loop(0, zb * nvec, _zero, 0)

        def _zcopy(k, _):
            pltpu.sync_copy(zero_v, acc_sh.at[pl.ds(sid * rpt + k * zb, zb)])
            return 0

        lax.fori_loop(0, rpt // zb, _zcopy, 0)
        plsc.subcore_barrier()

        def block(b, _):
            pltpu.sync_copy(src_hbm.at[wid, b], src_v)
            pltpu.sync_copy(dst_hbm.at[wid, b], dst_v)
            pltpu.sync_copy(ew_hbm.at[wid, b], ew_v)

            def chunk(j, _):
                pltpu.async_copy(h_hbm.at[src_v.at[j]], rows_v, sem).wait()

                def group(g, _):
                    ew16 = ew_v[j, pl.ds(g * LANES, LANES)]
                    for l in range(LANES):
                        s = ew16[l]
                        e = g * LANES + l
                        for k in range(nvec):
                            sl = pl.ds(k * LANES, LANES)
                            rows_v[e, sl] = rows_v[e, sl] * s
                    return 0

                lax.fori_loop(0, c // LANES, group, 0)
                pltpu.sync_copy(rows_v, acc_sh.at[dst_v.at[j]], add=True)
                return 0

            lax.fori_loop(0, sb, chunk, 0)
            return 0

        lax.fori_loop(0, nb, block, 0)
        plsc.subcore_barrier()
        pltpu.sync_copy(
            acc_sh.at[pl.ds(sid * rpt, rpt)],
            out_hbm.at[cid].at[pl.ds(sid * rpt, rpt)],
        )

    return agg_kernel


_ROWS = 1000


def _tc_first(p, dp, W, b):
    n = 10000
    d = p.shape[2]
    dh = W.shape[1]
    grid = n // _ROWS

    def body(p_ref, dp_ref, w_ref, b_ref, deg_ref, h_ref):
        deg = jnp.maximum(dp_ref[0, :, 0:1] + dp_ref[1, :, 0:1], 1e-6)
        deg_ref[...] = deg
        agg = (p_ref[0] + p_ref[1]) / deg
        h_ref[...] = jax.nn.gelu(
            jnp.dot(agg, w_ref[...], preferred_element_type=jnp.float32) + b_ref[...]
        )

    return pl.pallas_call(
        body,
        grid=(grid,),
        in_specs=[
            pl.BlockSpec((2, _ROWS, d), lambda i: (0, i, 0)),
            pl.BlockSpec((2, _ROWS, LANES), lambda i: (0, i, 0)),
            pl.BlockSpec(W.shape, lambda i: (0, 0)),
            pl.BlockSpec((1, dh), lambda i: (0, 0)),
        ],
        out_specs=[
            pl.BlockSpec((_ROWS, 1), lambda i: (i, 0)),
            pl.BlockSpec((_ROWS, dh), lambda i: (i, 0)),
        ],
        out_shape=[
            jax.ShapeDtypeStruct((n, 1), jnp.float32),
            jax.ShapeDtypeStruct((n, dh), jnp.float32),
        ],
    )(p, dp, W, b)


def _tc_layer(p, deg, W, b):
    n = deg.shape[0]
    d = p.shape[2]
    dh = W.shape[1]
    grid = n // _ROWS

    def body(p_ref, deg_ref, w_ref, b_ref, aggn_ref, h_ref):
        agg = (p_ref[0] + p_ref[1]) / deg_ref[...]
        aggn_ref[...] = agg
        h_ref[...] = jax.nn.gelu(
            jnp.dot(agg, w_ref[...], preferred_element_type=jnp.float32) + b_ref[...]
        )

    return pl.pallas_call(
        body,
        grid=(grid,),
        in_specs=[
            pl.BlockSpec((2, _ROWS, d), lambda i: (0, i, 0)),
            pl.BlockSpec((_ROWS, 1), lambda i: (i, 0)),
            pl.BlockSpec(W.shape, lambda i: (0, 0)),
            pl.BlockSpec((1, dh), lambda i: (0, 0)),
        ],
        out_specs=[
            pl.BlockSpec((_ROWS, d), lambda i: (i, 0)),
            pl.BlockSpec((_ROWS, dh), lambda i: (i, 0)),
        ],
        out_shape=[
            jax.ShapeDtypeStruct((n, d), jnp.float32),
            jax.ShapeDtypeStruct((n, dh), jnp.float32),
        ],
    )(p, deg, W, b)


def _tc_dec(p, deg, skip_aggn, W_top, W_bot, b):
    n = deg.shape[0]
    d = p.shape[2]
    dh = W_top.shape[1]
    grid = n // _ROWS

    def body(p_ref, deg_ref, sk_ref, wt_ref, wb_ref, b_ref, h_ref):
        agg = (p_ref[0] + p_ref[1]) / deg_ref[...]
        acc = jnp.dot(agg, wt_ref[...], preferred_element_type=jnp.float32)
        acc = acc + jnp.dot(sk_ref[...], wb_ref[...], preferred_element_type=jnp.float32)
        h_ref[...] = jax.nn.gelu(acc + b_ref[...])

    return pl.pallas_call(
        body,
        grid=(grid,),
        in_specs=[
            pl.BlockSpec((2, _ROWS, d), lambda i: (0, i, 0)),
            pl.BlockSpec((_ROWS, 1), lambda i: (i, 0)),
            pl.BlockSpec((_ROWS, d), lambda i: (i, 0)),
            pl.BlockSpec(W_top.shape, lambda i: (0, 0)),
            pl.BlockSpec(W_bot.shape, lambda i: (0, 0)),
            pl.BlockSpec((1, dh), lambda i: (0, 0)),
        ],
        out_specs=pl.BlockSpec((_ROWS, dh), lambda i: (i, 0)),
        out_shape=jax.ShapeDtypeStruct((n, dh), jnp.float32),
    )(p, deg, skip_aggn, W_top, W_bot, b)


def _tc_final(h, W, b):
    n, d = h.shape
    do = W.shape[1]
    grid = n // _ROWS

    def body(h_ref, w_ref, b_ref, o_ref):
        o_ref[...] = (
            jnp.dot(h_ref[...], w_ref[...], preferred_element_type=jnp.float32)
            + b_ref[...]
        )

    return pl.pallas_call(
        body,
        grid=(grid,),
        in_specs=[
            pl.BlockSpec((_ROWS, d), lambda i: (i, 0)),
            pl.BlockSpec(W.shape, lambda i: (0, 0)),
            pl.BlockSpec((1, do), lambda i: (0, 0)),
        ],
        out_specs=pl.BlockSpec((_ROWS, do), lambda i: (i, 0)),
        out_shape=jax.ShapeDtypeStruct((n, do), jnp.float32),
    )(h, W, b)


def kernel(x, edge_index, edge_weight, W_in, b_in, W_enc, b_enc,
           W_bot, b_bot, W_dec, b_dec, W_out, b_out):
    n, d = x.shape
    e = edge_weight.shape[0]
    e_blk = NW * NB * SB * CHUNK
    e_pad = ((e + e_blk - 1) // e_blk) * e_blk
    n_pad = ((n + 8 * NS - 1) // (8 * NS)) * (8 * NS)

    pad = [(0, e_pad - e)]
    src = jnp.pad(edge_index[0].astype(jnp.int32), pad).reshape(NW, NB, SB, CHUNK)
    dst = jnp.pad(edge_index[1].astype(jnp.int32), pad).reshape(NW, NB, SB, CHUNK)
    ewr = jnp.pad(edge_weight.astype(jnp.float32), pad).reshape(NW, NB, SB, CHUNK)

    deg_kernel = _make_deg_kernel(n_pad, NB, SB, CHUNK)
    agg_kernel = _make_agg_kernel(n_pad, d, NB, SB, CHUNK)

    dp = deg_kernel(dst, ewr)
    p = agg_kernel(x, src, dst, ewr)
    deg, h = _tc_first(p, dp, W_in, b_in.reshape(1, -1))

    n_layers = W_enc.shape[0]
    skip_aggs = {}
    for i in range(n_layers):
        p = agg_kernel(h, src, dst, ewr)
        aggn, h = _tc_layer(p, deg, W_enc[i], b_enc[i].reshape(1, -1))
        if i >= 1:
            skip_aggs[i - 1] = aggn

    p = agg_kernel(h, src, dst, ewr)
    aggn, h = _tc_layer(p, deg, W_bot, b_bot.reshape(1, -1))
    skip_aggs[n_layers - 1] = aggn

    for i in range(n_layers):
        p = agg_kernel(h, src, dst, ewr)
        h = _tc_dec(
            p, deg, skip_aggs[n_layers - 1 - i],
            W_dec[i][:d], W_dec[i][d:], b_dec[i].reshape(1, -1),
        )

    return _tc_final(h, W_out, b_out.reshape(1, -1))


# speedup vs baseline: 2.9549x; 1.0050x over previous
import functools

import jax
import jax.numpy as jnp
from jax import lax
from jax.experimental import pallas as pl
from jax.experimental.pallas import tpu as pltpu
from jax.experimental.pallas import tpu_sc as plsc

NC = 2
NS = 16
LANES = 16
NW = NC * NS

CHUNK = 128
SB = 16
NB = 5


def _sc_mesh():
    return plsc.VectorSubcoreMesh(
        core_axis_name="c", subcore_axis_name="s", num_cores=NC, num_subcores=NS
    )


@functools.lru_cache(maxsize=None)
def _make_deg_kernel(n_pad, nb, sb, c):
    rpt = n_pad // NS
    zb = 8

    @functools.partial(
        pl.kernel,
        out_type=jax.ShapeDtypeStruct((NC, n_pad, LANES), jnp.float32),
        mesh=_sc_mesh(),
        scratch_types=[
            pltpu.VMEM((sb, c), jnp.int32),
            pltpu.VMEM((sb, c), jnp.float32),
            pltpu.VMEM((c, LANES), jnp.float32),
            pltpu.VMEM((zb, LANES), jnp.float32),
            pltpu.VMEM_SHARED((n_pad, LANES), jnp.float32),
            pltpu.SemaphoreType.DMA,
        ],
    )
    def deg_kernel(dst_hbm, ew_hbm, out_hbm, dst_v, ew_v, rep_v, zero_v, acc_sh, sem):
        del sem
        cid = lax.axis_index("c")
        sid = lax.axis_index("s")
        wid = sid * NC + cid

        def _zero(i, _):
            zero_v[i] = jnp.zeros((LANES,), jnp.float32)
            return 0

        lax.fori_loop(0, zb, _zero, 0)

        def _zcopy(k, _):
            pltpu.sync_copy(zero_v, acc_sh.at[pl.ds(sid * rpt + k * zb, zb)])
            return 0

        lax.fori_loop(0, rpt // zb, _zcopy, 0)
        plsc.subcore_barrier()

        def block(b, _):
            pltpu.sync_copy(dst_hbm.at[wid, b], dst_v)
            pltpu.sync_copy(ew_hbm.at[wid, b], ew_v)

            def chunk(j, _):
                def group(g, _):
                    ew16 = ew_v[j, pl.ds(g * LANES, LANES)]
                    for l in range(LANES):
                        rep_v[g * LANES + l] = jnp.full((LANES,), ew16[l], jnp.float32)
                    return 0

                lax.fori_loop(0, c // LANES, group, 0)
                pltpu.sync_copy(rep_v, acc_sh.at[dst_v.at[j]], add=True)
                return 0

            lax.fori_loop(0, sb, chunk, 0)
            return 0

        lax.fori_loop(0, nb, block, 0)
        plsc.subcore_barrier()
        pltpu.sync_copy(
            acc_sh.at[pl.ds(sid * rpt, rpt)],
            out_hbm.at[cid].at[pl.ds(sid * rpt, rpt)],
        )

    return deg_kernel


@functools.lru_cache(maxsize=None)
def _make_agg_kernel(n_pad, d, nb, sb, c):
    rpt = n_pad // NS
    zb = 32

    @functools.partial(
        pl.kernel,
        out_type=jax.ShapeDtypeStruct((NC, n_pad, d), jnp.float32),
        mesh=_sc_mesh(),
        scratch_types=[
            pltpu.VMEM((sb, c), jnp.int32),
            pltpu.VMEM((sb, c), jnp.int32),
            pltpu.VMEM((sb, c), jnp.float32),
            pltpu.VMEM((2, c, d), jnp.float32),
            pltpu.VMEM((zb, d), jnp.float32),
            pltpu.VMEM_SHARED((n_pad, d), jnp.float32),
            pltpu.SemaphoreType.DMA,
            pltpu.SemaphoreType.DMA,
        ],
    )
    def agg_kernel(
        h_hbm, src_hbm, dst_hbm, ew_hbm, out_hbm,
        src_v, dst_v, ew_v, rows_v, zero_v, acc_sh, sem_g, sem_s,
    ):
        cid = lax.axis_index("c")
        sid = lax.axis_index("s")
        wid = sid * NC + cid
        nvec = d // LANES

        def _zero(i, _):
            r = i // nvec
            col = (i % nvec) * LANES
            zero_v[r, pl.ds(col, LANES)] = jnp.zeros((LANES,), jnp.float32)
            return 0

        lax.fori_loop(0, zb * nvec, _zero, 0)

        def _zcopy(k, _):
            pltpu.sync_copy(zero_v, acc_sh.at[pl.ds(sid * rpt + k * zb, zb)])
            return 0

        lax.fori_loop(0, rpt // zb, _zcopy, 0)
        plsc.subcore_barrier()

        def block(b, _):
            pltpu.sync_copy(src_hbm.at[wid, b], src_v)
            pltpu.sync_copy(dst_hbm.at[wid, b], dst_v)
            pltpu.sync_copy(ew_hbm.at[wid, b], ew_v)
            pltpu.async_copy(h_hbm.at[src_v.at[0]], rows_v.at[0], sem_g)

            def chunk(j, _):
                cur = lax.rem(j, 2)
                nxt = 1 - cur
                pltpu.make_async_copy(
                    h_hbm.at[src_v.at[j]], rows_v.at[cur], sem_g
                ).wait()

                @pl.when(j >= 1)
                def _():
                    pltpu.make_async_copy(
                        rows_v.at[nxt], acc_sh.at[dst_v.at[j - 1]], sem_s
                    ).wait()

                @pl.when(j + 1 < sb)
                def _():
                    pltpu.async_copy(
                        h_hbm.at[src_v.at[j + 1]], rows_v.at[nxt], sem_g
                    )

                def group(g, _):
                    ew16 = ew_v[j, pl.ds(g * LANES, LANES)]
                    for l in range(LANES):
                        s = ew16[l]
                        e = g * LANES + l
                        for k in range(nvec):
                            sl = pl.ds(k * LANES, LANES)
                            rows_v[cur, e, sl] = rows_v[cur, e, sl] * s
                    return 0

                lax.fori_loop(0, c // LANES, group, 0)
                pltpu.async_copy(
                    rows_v.at[cur], acc_sh.at[dst_v.at[j]], sem_s, add=True
                )
                return 0

            lax.fori_loop(0, sb, chunk, 0)
            pltpu.make_async_copy(
                rows_v.at[(sb - 1) % 2], acc_sh.at[dst_v.at[sb - 1]], sem_s
            ).wait()
            return 0

        lax.fori_loop(0, nb, block, 0)
        plsc.subcore_barrier()
        pltpu.sync_copy(
            acc_sh.at[pl.ds(sid * rpt, rpt)],
            out_hbm.at[cid].at[pl.ds(sid * rpt, rpt)],
        )

    return agg_kernel


_ROWS = 1000


def _tc_first(p, dp, W, b):
    n = 10000
    d = p.shape[2]
    dh = W.shape[1]
    grid = n // _ROWS

    def body(p_ref, dp_ref, w_ref, b_ref, deg_ref, h_ref):
        deg = jnp.maximum(dp_ref[0, :, 0:1] + dp_ref[1, :, 0:1], 1e-6)
        deg_ref[...] = deg
        agg = (p_ref[0] + p_ref[1]) / deg
        h_ref[...] = jax.nn.gelu(
            jnp.dot(agg, w_ref[...], preferred_element_type=jnp.float32) + b_ref[...]
        )

    return pl.pallas_call(
        body,
        grid=(grid,),
        in_specs=[
            pl.BlockSpec((2, _ROWS, d), lambda i: (0, i, 0)),
            pl.BlockSpec((2, _ROWS, LANES), lambda i: (0, i, 0)),
            pl.BlockSpec(W.shape, lambda i: (0, 0)),
            pl.BlockSpec((1, dh), lambda i: (0, 0)),
        ],
        out_specs=[
            pl.BlockSpec((_ROWS, 1), lambda i: (i, 0)),
            pl.BlockSpec((_ROWS, dh), lambda i: (i, 0)),
        ],
        out_shape=[
            jax.ShapeDtypeStruct((n, 1), jnp.float32),
            jax.ShapeDtypeStruct((n, dh), jnp.float32),
        ],
    )(p, dp, W, b)


def _tc_layer(p, deg, W, b):
    n = deg.shape[0]
    d = p.shape[2]
    dh = W.shape[1]
    grid = n // _ROWS

    def body(p_ref, deg_ref, w_ref, b_ref, aggn_ref, h_ref):
        agg = (p_ref[0] + p_ref[1]) / deg_ref[...]
        aggn_ref[...] = agg
        h_ref[...] = jax.nn.gelu(
            jnp.dot(agg, w_ref[...], preferred_element_type=jnp.float32) + b_ref[...]
        )

    return pl.pallas_call(
        body,
        grid=(grid,),
        in_specs=[
            pl.BlockSpec((2, _ROWS, d), lambda i: (0, i, 0)),
            pl.BlockSpec((_ROWS, 1), lambda i: (i, 0)),
            pl.BlockSpec(W.shape, lambda i: (0, 0)),
            pl.BlockSpec((1, dh), lambda i: (0, 0)),
        ],
        out_specs=[
            pl.BlockSpec((_ROWS, d), lambda i: (i, 0)),
            pl.BlockSpec((_ROWS, dh), lambda i: (i, 0)),
        ],
        out_shape=[
            jax.ShapeDtypeStruct((n, d), jnp.float32),
            jax.ShapeDtypeStruct((n, dh), jnp.float32),
        ],
    )(p, deg, W, b)


def _tc_dec(p, deg, skip_aggn, W_top, W_bot, b):
    n = deg.shape[0]
    d = p.shape[2]
    dh = W_top.shape[1]
    grid = n // _ROWS

    def body(p_ref, deg_ref, sk_ref, wt_ref, wb_ref, b_ref, h_ref):
        agg = (p_ref[0] + p_ref[1]) / deg_ref[...]
        acc = jnp.dot(agg, wt_ref[...], preferred_element_type=jnp.float32)
        acc = acc + jnp.dot(sk_ref[...], wb_ref[...], preferred_element_type=jnp.float32)
        h_ref[...] = jax.nn.gelu(acc + b_ref[...])

    return pl.pallas_call(
        body,
        grid=(grid,),
        in_specs=[
            pl.BlockSpec((2, _ROWS, d), lambda i: (0, i, 0)),
            pl.BlockSpec((_ROWS, 1), lambda i: (i, 0)),
            pl.BlockSpec((_ROWS, d), lambda i: (i, 0)),
            pl.BlockSpec(W_top.shape, lambda i: (0, 0)),
            pl.BlockSpec(W_bot.shape, lambda i: (0, 0)),
            pl.BlockSpec((1, dh), lambda i: (0, 0)),
        ],
        out_specs=pl.BlockSpec((_ROWS, dh), lambda i: (i, 0)),
        out_shape=jax.ShapeDtypeStruct((n, dh), jnp.float32),
    )(p, deg, skip_aggn, W_top, W_bot, b)


def _tc_final(h, W, b):
    n, d = h.shape
    do = W.shape[1]
    grid = n // _ROWS

    def body(h_ref, w_ref, b_ref, o_ref):
        o_ref[...] = (
            jnp.dot(h_ref[...], w_ref[...], preferred_element_type=jnp.float32)
            + b_ref[...]
        )

    return pl.pallas_call(
        body,
        grid=(grid,),
        in_specs=[
            pl.BlockSpec((_ROWS, d), lambda i: (i, 0)),
            pl.BlockSpec(W.shape, lambda i: (0, 0)),
            pl.BlockSpec((1, do), lambda i: (0, 0)),
        ],
        out_specs=pl.BlockSpec((_ROWS, do), lambda i: (i, 0)),
        out_shape=jax.ShapeDtypeStruct((n, do), jnp.float32),
    )(h, W, b)


def kernel(x, edge_index, edge_weight, W_in, b_in, W_enc, b_enc,
           W_bot, b_bot, W_dec, b_dec, W_out, b_out):
    n, d = x.shape
    e = edge_weight.shape[0]
    e_blk = NW * NB * SB * CHUNK
    e_pad = ((e + e_blk - 1) // e_blk) * e_blk
    n_pad = ((n + 8 * NS - 1) // (8 * NS)) * (8 * NS)

    pad = [(0, e_pad - e)]
    src = jnp.pad(edge_index[0].astype(jnp.int32), pad).reshape(NW, NB, SB, CHUNK)
    dst = jnp.pad(edge_index[1].astype(jnp.int32), pad).reshape(NW, NB, SB, CHUNK)
    ewr = jnp.pad(edge_weight.astype(jnp.float32), pad).reshape(NW, NB, SB, CHUNK)

    deg_kernel = _make_deg_kernel(n_pad, NB, SB, CHUNK)
    agg_kernel = _make_agg_kernel(n_pad, d, NB, SB, CHUNK)

    dp = deg_kernel(dst, ewr)
    p = agg_kernel(x, src, dst, ewr)
    deg, h = _tc_first(p, dp, W_in, b_in.reshape(1, -1))

    n_layers = W_enc.shape[0]
    skip_aggs = {}
    for i in range(n_layers):
        p = agg_kernel(h, src, dst, ewr)
        aggn, h = _tc_layer(p, deg, W_enc[i], b_enc[i].reshape(1, -1))
        if i >= 1:
            skip_aggs[i - 1] = aggn

    p = agg_kernel(h, src, dst, ewr)
    aggn, h = _tc_layer(p, deg, W_bot, b_bot.reshape(1, -1))
    skip_aggs[n_layers - 1] = aggn

    for i in range(n_layers):
        p = agg_kernel(h, src, dst, ewr)
        h = _tc_dec(
            p, deg, skip_aggs[n_layers - 1 - i],
            W_dec[i][:d], W_dec[i][d:], b_dec[i].reshape(1, -1),
        )

    return _tc_final(h, W_out, b_out.reshape(1, -1))
